# baseline (speedup 1.0000x reference)
import functools

import jax
import jax.numpy as jnp
from jax import lax
from jax.experimental import pallas as pl
from jax.experimental.pallas import tpu as pltpu
from jax.experimental.pallas import tpu_sc as plsc

B = 130000
V = 1000000
H = 64
L = 16
NC, NS = 2, 16
NW = NC * NS
BP = 131072
BW = BP // NW
CH = 128
NCHUNK = BW // CH
NB = 2
NGRP = NCHUNK // NB
RW = BW // CH

_CPT = pltpu.CompilerParams(needs_layout_passes=False, use_tc_tiling_on_sc=True)
_MESH = dict(core_axis_name="c", subcore_axis_name="s")


def _rowgather(emb_h, idx, g, b, rows, sems):
    def sub(s, _):
        off = pl.multiple_of(s * L, L)
        iv = idx[g, pl.ds(off, L)]
        for j in range(L):
            r = iv[j]
            pltpu.make_async_copy(
                emb_h.at[pl.ds(r >> 1, 1)],
                rows[b].at[pl.ds(s * L + j, 1)], sems[b]).start()
        return 0
    lax.fori_loop(0, CH // L, sub, 0)


def _sc_wgather(wrd2d, wemb2):
    @functools.partial(
        pl.kernel,
        compiler_params=_CPT,
        out_type=jax.ShapeDtypeStruct((BP, 2 * H), jnp.float32),
        mesh=plsc.VectorSubcoreMesh(**_MESH),
        scratch_types=[
            pltpu.VMEM((NCHUNK, CH), jnp.int32),
            [pltpu.VMEM((CH, 2 * H), jnp.float32)] * NB,
            [pltpu.SemaphoreType.DMA] * NB,
            [pltpu.SemaphoreType.DMA] * NB,
        ],
    )
    def k(wrd_h, wemb_h, out_h, widx, wrows, gsems, osems):
        wid = lax.axis_index("s") * NC + lax.axis_index("c")
        r0 = pl.multiple_of(wid * RW, RW)
        i0 = pl.multiple_of(wid * BW, BW)

        pltpu.sync_copy(wrd_h.at[pl.ds(r0, RW)], widx)

        def gather_wait(b):
            pltpu.make_async_copy(
                wemb_h.at[pl.ds(0, CH)], wrows[b], gsems[b]).wait()

        def write_start(g, b):
            pltpu.make_async_copy(
                wrows[b], out_h.at[pl.ds(i0 + g * CH, CH)], osems[b]).start()

        def write_wait(b):
            pltpu.make_async_copy(
                wrows[b], out_h.at[pl.ds(0, CH)], osems[b]).wait()

        for b in range(NB):
            _rowgather(wemb_h, widx, b, b, wrows, gsems)

        def grp_body(grp, _):
            for b in range(NB):
                g = grp * NB + b
                gather_wait(b)

                @pl.when(grp > 0)
                def _():
                    write_wait(b)
                write_start(g, b)

                @pl.when(grp < NGRP - 1)
                def _():
                    _rowgather(wemb_h, widx, g + NB, b, wrows, gsems)
            return 0

        lax.fori_loop(0, NGRP, grp_body, 0)
        for b in range(NB):
            write_wait(b)

    return k(wrd2d, wemb2)


def _sc_cdots(wrd2d, cntxt2d, cemb2, wrows2d):
    @functools.partial(
        pl.kernel,
        compiler_params=_CPT,
        out_type=jax.ShapeDtypeStruct((BP // CH, CH), jnp.float32),
        mesh=plsc.VectorSubcoreMesh(**_MESH),
        scratch_types=[
            pltpu.VMEM((NCHUNK, CH), jnp.int32),
            pltpu.VMEM((NCHUNK, CH), jnp.int32),
            [pltpu.VMEM((CH, 2 * H), jnp.float32)] * NB,
            [pltpu.VMEM((CH, 2 * H), jnp.float32)] * NB,
            pltpu.VMEM((NCHUNK, CH), jnp.float32),
            [pltpu.SemaphoreType.DMA] * NB,
            [pltpu.SemaphoreType.DMA] * NB,
        ],
    )
    def k(wrd_h, cx_h, cemb_h, wflat_h, out_h,
          widx, cidx, crows, wbufs, dots, csems, wsems):
        wid = lax.axis_index("s") * NC + lax.axis_index("c")
        r0 = pl.multiple_of(wid * RW, RW)
        i0 = pl.multiple_of(wid * BW, BW)

        pltpu.sync_copy(wrd_h.at[pl.ds(r0, RW)], widx)
        pltpu.sync_copy(cx_h.at[pl.ds(r0, RW)], cidx)

        def chunk_start(g, b):
            pltpu.make_async_copy(
                wflat_h.at[pl.ds(i0 + g * CH, CH)],
                wbufs[b], wsems[b]).start()
            _rowgather(cemb_h, cidx, g, b, crows, csems)

        def chunk_wait(b):
            pltpu.make_async_copy(
                wflat_h.at[pl.ds(0, CH)], wbufs[b], wsems[b]).wait()
            pltpu.make_async_copy(
                cemb_h.at[pl.ds(0, CH)], crows[b], csems[b]).wait()

        for b in range(NB):
            chunk_start(b, b)

        lanes = lax.iota(jnp.int32, L)

        def dotgroup(wb, cr, iv, wpar, cpar):
            accs = [jnp.zeros((L,), jnp.float32) for _ in range(4)]
            for h in range(H):
                wv = plsc.load_gather(wb, [iv, wpar + h])
                cv = plsc.load_gather(cr, [iv, cpar + h])
                accs[h % 4] = accs[h % 4] + wv * cv
            return (accs[0] + accs[1]) + (accs[2] + accs[3])

        def compute(g, b):
            def body(j, _):
                iv = jnp.full((L,), j * L, jnp.int32) + lanes
                off = pl.multiple_of(j * L, L)
                wpar = (widx[g, pl.ds(off, L)] & 1) * H
                cpar = (cidx[g, pl.ds(off, L)] & 1) * H
                dots[g, pl.ds(off, L)] = dotgroup(
                    wbufs[b], crows[b], iv, wpar, cpar)
                return 0
            lax.fori_loop(0, CH // L, body, 0)

        def grp_body(grp, _):
            for b in range(NB):
                g = grp * NB + b
                chunk_wait(b)
                compute(g, b)

                @pl.when(grp < NGRP - 1)
                def _():
                    chunk_start(g + NB, b)
            return 0

        lax.fori_loop(0, NGRP, grp_body, 0)

        pltpu.sync_copy(dots, out_h.at[pl.ds(r0, RW)])

    return k(wrd2d, cntxt2d, cemb2, wrows2d)


def _tc_loss(dots2d, labels2d):
    def body(d_ref, l_ref, o_ref):
        x = d_ref[...] * l_ref[...]
        r = lax.broadcasted_iota(jnp.int32, x.shape, 0)
        c = lax.broadcasted_iota(jnp.int32, x.shape, 1)
        valid = (r * x.shape[1] + c) < B
        ls = jnp.where(valid, jax.nn.log_sigmoid(x), 0.0)
        o_ref[0, 0] = jnp.sum(ls) * (-1.0 / B)

    out = pl.pallas_call(
        body,
        out_shape=jax.ShapeDtypeStruct((1, 1), jnp.float32),
        out_specs=pl.BlockSpec(memory_space=pltpu.SMEM),
    )(dots2d, labels2d)
    return out[0, 0]


def kernel(wrd, cntxt, labels, word_emb, context_emb):
    pad = BP - B
    wrd_p = jnp.concatenate(
        [wrd.reshape(-1), jnp.zeros((pad,), jnp.int32)]).reshape(BP // CH, CH)
    cx_p = jnp.concatenate(
        [cntxt.reshape(-1), jnp.zeros((pad,), jnp.int32)]).reshape(BP // CH, CH)
    lab_p = jnp.concatenate(
        [labels.reshape(-1), jnp.zeros((pad,), jnp.float32)]).reshape(BP // CH, CH)
    wemb2 = word_emb.reshape(V // 2, 2 * H)
    cemb2 = context_emb.reshape(V // 2, 2 * H)
    wrows = _sc_wgather(wrd_p, wemb2)
    dots = _sc_cdots(wrd_p, cx_p, cemb2, wrows)
    return _tc_loss(dots, lab_p)


# speedup vs baseline: 1.6973x; 1.6973x over previous
import functools

import jax
import jax.numpy as jnp
from jax import lax
from jax.experimental import pallas as pl
from jax.experimental.pallas import tpu as pltpu
from jax.experimental.pallas import tpu_sc as plsc

B = 130000
V = 1000000
H = 64
L = 16
NC, NS = 2, 16
NW = NC * NS
BP = 131072
BW = BP // NW
CH = 128
NCHUNK = BW // CH
NB = 2
NGRP = NCHUNK // NB
RW = BW // CH


def _sc_dots(wrd2d, cntxt2d, word_emb, context_emb):
    mesh = plsc.VectorSubcoreMesh(core_axis_name="c", subcore_axis_name="s")

    @functools.partial(
        pl.kernel,
        compiler_params=pltpu.CompilerParams(
            needs_layout_passes=False, use_tc_tiling_on_sc=True),
        out_type=jax.ShapeDtypeStruct((BP // CH, CH), jnp.float32),
        mesh=mesh,
        scratch_types=[
            pltpu.VMEM((NCHUNK, CH), jnp.int32),
            pltpu.VMEM((NCHUNK, CH), jnp.int32),
            [pltpu.VMEM((CH, H), jnp.float32)] * NB,
            [pltpu.VMEM((CH, H), jnp.float32)] * NB,
            pltpu.VMEM((NCHUNK, CH), jnp.float32),
            [pltpu.SemaphoreType.DMA] * NB,
        ],
    )
    def k(wrd_h, cx_h, wemb_h, cemb_h, out_h,
          widx, cidx, wrows, crows, dots, gsems):
        wid = lax.axis_index("s") * NC + lax.axis_index("c")
        r0 = pl.multiple_of(wid * RW, RW)

        pltpu.sync_copy(wrd_h.at[pl.ds(r0, RW)], widx)
        pltpu.sync_copy(cx_h.at[pl.ds(r0, RW)], cidx)

        def chunk_start(g, b):
            def sub(s, _):
                off = pl.multiple_of(s * L, L)
                ivw = widx[g, pl.ds(off, L)]
                ivc = cidx[g, pl.ds(off, L)]
                for j in range(L):
                    pltpu.make_async_copy(
                        wemb_h.at[pl.ds(ivw[j], 1)],
                        wrows[b].at[pl.ds(s * L + j, 1)], gsems[b]).start()
                    pltpu.make_async_copy(
                        cemb_h.at[pl.ds(ivc[j], 1)],
                        crows[b].at[pl.ds(s * L + j, 1)], gsems[b]).start()
                return 0
            lax.fori_loop(0, CH // L, sub, 0)

        def chunk_wait(b):
            pltpu.make_async_copy(
                wemb_h.at[pl.ds(0, CH)], wrows[b], gsems[b]).wait()
            pltpu.make_async_copy(
                cemb_h.at[pl.ds(0, CH)], crows[b], gsems[b]).wait()

        for b in range(NB):
            chunk_start(b, b)

        lanes = lax.iota(jnp.int32, L)
        lastlane = lanes == (L - 1)
        gsplat_cache = {}

        def dotitem(wr, cr, g, i):
            parts = []
            for q in range(4):
                sl = pl.ds(q * L, L)
                parts.append(wr[i, sl] * cr[i, sl])
            t = (parts[0] + parts[1]) + (parts[2] + parts[3])
            c = plsc.cumsum(t)
            plsc.store_scatter(
                dots, [gsplat_cache[0], jnp.full((L,), i, jnp.int32)],
                c, mask=lastlane)

        def compute(g, b):
            gsplat_cache[0] = jnp.full((L,), g, jnp.int32)

            def body(it, _):
                i = it * 2
                dotitem(wrows[b], crows[b], g, i)
                dotitem(wrows[b], crows[b], g, i + 1)
                return 0
            lax.fori_loop(0, CH // 2, body, 0)

        def grp_body(grp, _):
            for b in range(NB):
                g = grp * NB + b
                chunk_wait(b)
                compute(g, b)

                @pl.when(grp < NGRP - 1)
                def _():
                    chunk_start(g + NB, b)
            return 0

        lax.fori_loop(0, NGRP, grp_body, 0)

        pltpu.sync_copy(dots, out_h.at[pl.ds(r0, RW)])

    return k(wrd2d, cntxt2d, word_emb, context_emb)


def _tc_loss(dots2d, labels2d):
    def body(d_ref, l_ref, o_ref):
        x = d_ref[...] * l_ref[...]
        r = lax.broadcasted_iota(jnp.int32, x.shape, 0)
        c = lax.broadcasted_iota(jnp.int32, x.shape, 1)
        valid = (r * x.shape[1] + c) < B
        ls = jnp.where(valid, jax.nn.log_sigmoid(x), 0.0)
        o_ref[0, 0] = jnp.sum(ls) * (-1.0 / B)

    out = pl.pallas_call(
        body,
        out_shape=jax.ShapeDtypeStruct((1, 1), jnp.float32),
        out_specs=pl.BlockSpec(memory_space=pltpu.SMEM),
    )(dots2d, labels2d)
    return out[0, 0]


def kernel(wrd, cntxt, labels, word_emb, context_emb):
    pad = BP - B
    wrd_p = jnp.concatenate(
        [wrd.reshape(-1), jnp.zeros((pad,), jnp.int32)]).reshape(BP // CH, CH)
    cx_p = jnp.concatenate(
        [cntxt.reshape(-1), jnp.zeros((pad,), jnp.int32)]).reshape(BP // CH, CH)
    lab_p = jnp.concatenate(
        [labels.reshape(-1), jnp.zeros((pad,), jnp.float32)]).reshape(BP // CH, CH)
    dots = _sc_dots(wrd_p, cx_p, word_emb, context_emb)
    return _tc_loss(dots, lab_p)
